# Initial kernel scaffold; baseline (speedup 1.0000x reference)
import jax
import jax.numpy as jnp
from jax.experimental import pallas as pl


def kernel(data, params):
    raise NotImplementedError("write your pallas kernel here")


# speedup vs baseline: 3.8330x; 3.8330x over previous
import numpy as np
import jax
import jax.numpy as jnp
from jax.experimental import pallas as pl
from jax.experimental.pallas import tpu as pltpu

B, N = 2, 4096
KNN = 16
RT = 256
M = 2025
MP = 2048
EPS = 1e-5
P_MM = jax.lax.Precision.DEFAULT
P_EXACT = jax.lax.Precision.HIGHEST

_xx = np.linspace(-40, 40, 45, dtype=np.float32)
_yy = np.linspace(-60, 60, 45, dtype=np.float32)
_grid_np = np.stack(np.meshgrid(_xx, _yy)).reshape(2, -1)
_perm = np.random.RandomState(0).permutation(M)[: min(N, M)]
_gridp = _grid_np[:, _perm].T
_G_np = np.zeros((2 * MP, 8), dtype=np.float32)
_G_np[:M, :2] = _gridp
_G_np[MP : MP + M, :2] = _gridp
_RM_np = np.zeros((2 * MP, 1), dtype=np.float32)
_RM_np[:M] = 1.0
_RM_np[MP : MP + M] = 1.0


def _dist_tile(xr, xt):
    sqr = jnp.sum(xr * xr, axis=1, keepdims=True)
    sqa = jnp.sum(xt * xt, axis=0, keepdims=True)
    return (sqr - 2.0 * jnp.dot(xr, xt, preferred_element_type=jnp.float32,
                                precision=P_MM) + sqa)


def _knn_idx_body(xr_ref, xt_ref, out_ref, d_ref, i_ref):
    d_ref[...] = _dist_tile(xr_ref[0], xt_ref[0])

    def step(t, carry):
        iota = jax.lax.broadcasted_iota(jnp.int32, (RT, N), 1)
        d = d_ref[...]
        mval = jnp.min(d, axis=1, keepdims=True)
        first = jnp.min(jnp.where(d <= mval, iota, N), axis=1, keepdims=True)
        i_ref[t] = first
        d_ref[...] = jnp.where(iota == first, jnp.inf, d)
        return carry

    jax.lax.fori_loop(0, KNN, step, 0)
    out_ref[0] = jnp.concatenate([i_ref[t] for t in range(KNN)], axis=1)


def _knn_idx(x, xt, C):
    return pl.pallas_call(
        _knn_idx_body,
        grid=(B, N // RT),
        in_specs=[
            pl.BlockSpec((1, RT, C), lambda b, r: (b, r, 0)),
            pl.BlockSpec((1, C, N), lambda b, r: (b, 0, 0)),
        ],
        out_specs=pl.BlockSpec((1, RT, KNN), lambda b, r: (b, r, 0)),
        out_shape=jax.ShapeDtypeStruct((B, N, KNN), jnp.int32),
        scratch_shapes=[pltpu.VMEM((RT, N), jnp.float32),
                        pltpu.VMEM((KNN, RT, 1), jnp.int32)],
    )(x, xt)


def _conv_stats_body(x_ref, w_ref, b_ref, z_ref, st_ref):
    r = pl.program_id(0)
    z = (jnp.dot(x_ref[...], w_ref[...], preferred_element_type=jnp.float32,
                 precision=P_MM) + b_ref[0:1])
    z_ref[...] = z

    @pl.when(r == 0)
    def _():
        st_ref[...] = jnp.zeros_like(st_ref)

    st_ref[0:1] += jnp.sum(z, axis=0, keepdims=True)
    st_ref[1:2] += jnp.sum(z * z, axis=0, keepdims=True)


def _conv_stats(x, wt, bvec, tile):
    rows, Co = x.shape[0], wt.shape[1]
    return pl.pallas_call(
        _conv_stats_body,
        grid=(rows // tile,),
        in_specs=[
            pl.BlockSpec((tile, x.shape[1]), lambda r: (r, 0)),
            pl.BlockSpec(wt.shape, lambda r: (0, 0)),
            pl.BlockSpec((1, Co), lambda r: (0, 0)),
        ],
        out_specs=[
            pl.BlockSpec((tile, Co), lambda r: (r, 0)),
            pl.BlockSpec((8, Co), lambda r: (0, 0)),
        ],
        out_shape=[
            jax.ShapeDtypeStruct((rows, Co), jnp.float32),
            jax.ShapeDtypeStruct((8, Co), jnp.float32),
        ],
    )(x, wt, bvec[None, :])


def _conv4_body(z_ref, w_ref, aff_ref, y_ref, st_ref):
    b = pl.program_id(0)
    r = pl.program_id(1)
    u = jnp.maximum(
        aff_ref[2:3] * (z_ref[...] - aff_ref[0:1]) / jnp.sqrt(aff_ref[1:2] + EPS)
        + aff_ref[3:4], 0.0)
    y = (jnp.dot(u, w_ref[...], preferred_element_type=jnp.float32,
                 precision=P_MM) + aff_ref[4:5, 0:512])
    y_ref[...] = y

    @pl.when(jnp.logical_and(b == 0, r == 0))
    def _():
        st_ref[...] = jnp.full_like(st_ref, -jnp.inf)

    ymax = jnp.max(y, axis=0, keepdims=True)
    ymin = -jnp.min(y, axis=0, keepdims=True)

    @pl.when(b == 0)
    def _():
        st_ref[0:1] = jnp.maximum(st_ref[0:1], ymax)
        st_ref[2:3] = jnp.maximum(st_ref[2:3], ymin)

    @pl.when(b == 1)
    def _():
        st_ref[1:2] = jnp.maximum(st_ref[1:2], ymax)
        st_ref[3:4] = jnp.maximum(st_ref[3:4], ymin)


def _conv4_stats(z, wt, aff, tile):
    per_b = N // tile
    return pl.pallas_call(
        _conv4_body,
        grid=(B, per_b),
        in_specs=[
            pl.BlockSpec((tile, z.shape[1]), lambda b, r: (b * per_b + r, 0)),
            pl.BlockSpec(wt.shape, lambda b, r: (0, 0)),
            pl.BlockSpec(aff.shape, lambda b, r: (0, 0)),
        ],
        out_specs=[
            pl.BlockSpec((tile, 512), lambda b, r: (b * per_b + r, 0)),
            pl.BlockSpec((8, 512), lambda b, r: (0, 0)),
        ],
        out_shape=[
            jax.ShapeDtypeStruct((B * N, 512), jnp.float32),
            jax.ShapeDtypeStruct((8, 512), jnp.float32),
        ],
    )(z, wt, aff)


def _fold_stage(h, g, be, rm, cnt):
    mu = jnp.sum(h, axis=0, keepdims=True) / cnt
    hc = (h - mu) * rm
    var = jnp.sum(hc * hc, axis=0, keepdims=True) / cnt
    return jnp.maximum(g * hc / jnp.sqrt(var + EPS) + be, 0.0) * rm


def _decoder_body(g_ref, rm_ref, code_ref, wg1_ref, wc1_ref, w12_ref, w13_ref,
                  wr2_ref, wc2_ref, w22_ref, w23_ref, aff_ref, out_ref):
    cnt = float(2 * M)
    rm = rm_ref[...]
    code = code_ref[...]
    bsel = jax.lax.broadcasted_iota(jnp.int32, (2 * MP, 1), 0) >= MP

    def fold(x, wx_ref, wc_ref, w2_ref, w3_ref, a0, a3):
        bias = jnp.dot(code, wc_ref[...], preferred_element_type=jnp.float32,
                       precision=P_MM) + aff_ref[a0 : a0 + 1]
        h = jnp.dot(x, wx_ref[...], preferred_element_type=jnp.float32,
                    precision=P_MM)
        h = (h + jnp.where(bsel, bias[1:2], bias[0:1])) * rm
        h = _fold_stage(h, aff_ref[a0 + 1 : a0 + 2], aff_ref[a0 + 2 : a0 + 3], rm, cnt)
        h = (jnp.dot(h, w2_ref[...], preferred_element_type=jnp.float32,
                     precision=P_MM) + aff_ref[a0 + 3 : a0 + 4]) * rm
        h = _fold_stage(h, aff_ref[a0 + 4 : a0 + 5], aff_ref[a0 + 5 : a0 + 6], rm, cnt)
        return (jnp.dot(h, w3_ref[...], preferred_element_type=jnp.float32,
                        precision=P_MM) + aff_ref[a3 : a3 + 1, 0:128]) * rm

    r1 = fold(g_ref[...], wg1_ref, wc1_ref, w12_ref, w13_ref, 0, 12)
    out_ref[...] = fold(r1, wr2_ref, wc2_ref, w22_ref, w23_ref, 6, 13)


def _decoder(gmat, rmask, code, weights, aff):
    return pl.pallas_call(
        _decoder_body,
        out_shape=jax.ShapeDtypeStruct((2 * MP, 128), jnp.float32),
    )(gmat, rmask, code, *weights, aff)


def _c1d(x, W, b):
    return jnp.einsum('oi,bin->bon', W, x) + b[None, :, None]


def _bnx(x, g, b):
    mean = jnp.mean(x, axis=(0, 2), keepdims=True)
    var = jnp.var(x, axis=(0, 2), keepdims=True)
    return g[None, :, None] * (x - mean) / jnp.sqrt(var + EPS) + b[None, :, None]


def _aff_rows(p, names, width):
    rows = [jnp.pad(p[n], (0, width - p[n].shape[0])) for n in names]
    while len(rows) < 16:
        rows.append(jnp.zeros((width,), jnp.float32))
    return jnp.stack(rows)


def kernel(data, params):
    p = params
    data_pad = jnp.pad(data, ((0, 0), (0, 0), (0, 5)))
    data_t = jnp.swapaxes(data_pad, 1, 2)

    idx = _knn_idx(data_pad, data_t, 8)
    bi = jnp.arange(B)[:, None, None]
    kx = data[bi, idx, :]
    mean = jnp.mean(kx, axis=2, keepdims=True)
    kxc = kx - mean
    cov = jnp.einsum('bnkc,bnkd->bncd', kxc, kxc).reshape(B, N, -1)
    x = jnp.swapaxes(jnp.concatenate([data, cov], axis=2), 1, 2)
    x = jax.nn.relu(_bnx(_c1d(x, p['conv1_W'], p['conv1_b']), p['bn1_g'], p['bn1_be']))
    x = jax.nn.relu(_bnx(_c1d(x, p['conv2_W'], p['conv2_b']), p['bn2_g'], p['bn2_be']))
    x3t = jax.nn.relu(_bnx(_c1d(x, p['conv3_W'], p['conv3_b']), p['bn3_g'], p['bn3_be']))

    x3 = jnp.swapaxes(x3t, 1, 2)
    idx1 = _knn_idx(x3, x3t, 64)
    m1 = jax.lax.optimization_barrier(jnp.max(x3[bi, idx1, :], axis=2))
    hg1 = jax.lax.optimization_barrier(_c1d(jnp.swapaxes(m1, 1, 2), p['g1_W'], p['g1_b']))
    x4t = jax.nn.relu(_bnx(hg1, p['g1_g'], p['g1_be']))

    x4 = jnp.swapaxes(x4t, 1, 2)
    idx2 = _knn_idx(x4, x4t, 128)
    m2 = jax.lax.optimization_barrier(jnp.max(x4[bi, idx2, :], axis=2))

    z, st2 = _conv_stats(m2.reshape(B * N, 128), p['g2_W'].T, p['g2_b'], 1024)
    z3 = jnp.swapaxes(z.reshape(B, N, 1024), 1, 2)
    mu2 = jnp.mean(z3, axis=(0, 2))
    var2 = jnp.var(z3, axis=(0, 2))
    aff4 = jnp.stack([mu2, var2, p['g2_g'], p['g2_be'],
                      jnp.pad(p['conv4_b'], (0, 512))])
    y, st = _conv4_stats(z, p['conv4_W'].T, aff4, 1024)
    y3 = jnp.swapaxes(y.reshape(B, N, 512), 1, 2)
    mu4 = jnp.mean(y3, axis=(0, 2))
    var4 = jnp.var(y3, axis=(0, 2))

    def bn4(v):
        return p['bn4_g'] * (v - mu4) / jnp.sqrt(var4 + EPS) + p['bn4_be']

    pos = p['bn4_g'] > 0
    code = jnp.stack([
        jnp.where(pos, bn4(st[0]), bn4(-st[2])),
        jnp.where(pos, bn4(st[1]), bn4(-st[3])),
    ])

    gmat = jnp.asarray(_G_np)
    rmask = jnp.asarray(_RM_np)
    wg1 = jnp.pad(p['f1c1_W'][:, :2], ((0, 0), (0, 6))).T
    wc1 = p['f1c1_W'][:, 2:].T
    w13 = jnp.pad(p['f1c3_W'], ((0, 125), (0, 0))).T
    wr2 = jnp.pad(p['f2c1_W'][:, :3], ((0, 0), (0, 125))).T
    wc2 = p['f2c1_W'][:, 3:].T
    w23 = jnp.pad(p['f2c3_W'], ((0, 125), (0, 0))).T
    affd = _aff_rows(p, ['f1c1_b', 'f1bn1_g', 'f1bn1_be',
                         'f1c2_b', 'f1bn2_g', 'f1bn2_be',
                         'f2c1_b', 'f2bn1_g', 'f2bn1_be',
                         'f2c2_b', 'f2bn2_g', 'f2bn2_be',
                         'f1c3_b', 'f2c3_b'], 512)
    r2 = _decoder(gmat, rmask, code,
                  [wg1, wc1, p['f1c2_W'].T, w13,
                   wr2, wc2, p['f2c2_W'].T, w23], affd)
    return r2.reshape(B, MP, 128)[:, :M, :3]
